# baseline (speedup 1.0000x reference)
import functools

import jax
import jax.numpy as jnp
import numpy as np
from jax import lax
from jax.experimental import pallas as pl
from jax.experimental.pallas import tpu as pltpu
from jax.experimental.pallas import tpu_sc as plsc

VOCAB = 1000
D = 128
L = 200
B = 4096
OUT = L + 1

VPAD = 1024
NPLANE = VPAD // 128

NC = 2
NS = 16
LANES = 16
NW = NC * NS
SEQ_PER_W = B // NW
GROUPS = SEQ_PER_W // LANES
NBUF = 4


def _hist_body(x_hbm, counts_hbm, *refs):
    x_v = refs[0:GROUPS]
    counts_v = refs[GROUPS:GROUPS + NBUF]
    zeros_sp = refs[GROUPS + NBUF]
    n0 = GROUPS + NBUF + 1
    sx = refs[n0:n0 + GROUPS]
    sz = refs[n0 + GROUPS:n0 + GROUPS + NBUF]
    so = refs[n0 + GROUPS + NBUF:n0 + GROUPS + 2 * NBUF]

    cid = lax.axis_index("c")
    sid = lax.axis_index("s")
    wid = sid * NC + cid
    row_ids = lax.iota(jnp.int32, LANES)
    ones = jnp.ones((LANES,), jnp.float32)

    zslab = counts_v[0]

    def zero_body(it, zslab=zslab):
        j = lax.shift_right_logical(it, 3)
        c = lax.shift_left(lax.bitwise_and(it, 7), 4)
        zslab[0, j, pl.ds(c, LANES)] = jnp.zeros((LANES,), jnp.float32)

    plsc.parallel_loop(0, NPLANE * 8, step=1, unroll=8)(zero_body)
    pltpu.sync_copy(zslab.at[0], zeros_sp.at[sid])
    plsc.subcore_barrier()

    def x_copy(g):
        base = wid * SEQ_PER_W + g * LANES
        return pltpu.make_async_copy(
            x_hbm.at[pl.ds(base, LANES), :], x_v[g], sx[g])

    def z_copy(b):
        return pltpu.make_async_copy(zeros_sp, counts_v[b], sz[b])

    def out_copy(g, b, j):
        base = wid * SEQ_PER_W + g * LANES
        return pltpu.make_async_copy(
            counts_v[b].at[:, j, :],
            counts_hbm.at[j, pl.ds(base, LANES), :], so[b])

    for g in range(GROUPS):
        x_copy(g).start()
    z_copy(0).start()

    for g in range(GROUPS):
        b = g % NBUF
        if g + 1 < GROUPS:
            b2 = (g + 1) % NBUF
            if g + 1 >= NBUF:
                for j in range(NPLANE):
                    out_copy(g + 1 - NBUF, b2, j).wait()
            z_copy(b2).start()
        x_copy(g).wait()
        z_copy(b).wait()

        x_b = x_v[g]
        c_b = counts_v[b]

        def tok_body(l, x_b=x_b, c_b=c_b):
            col = jnp.full((LANES,), l, dtype=jnp.int32)
            toks = plsc.load_gather(x_b, [row_ids, col])
            plsc.addupdate_scatter(
                c_b,
                [row_ids, lax.shift_right_logical(toks, 7),
                 lax.bitwise_and(toks, 127)],
                ones)

        plsc.parallel_loop(0, L, step=1, unroll=8)(tok_body)

        for j in range(NPLANE):
            out_copy(g, b, j).start()
    for g in range(GROUPS - NBUF, GROUPS):
        for j in range(NPLANE):
            out_copy(g, g % NBUF, j).wait()


def _histogram(x):
    mesh = plsc.VectorSubcoreMesh(core_axis_name="c", subcore_axis_name="s")
    scratch = (
        [pltpu.VMEM((LANES, L), jnp.int32) for _ in range(GROUPS)]
        + [pltpu.VMEM((LANES, NPLANE, 128), jnp.float32) for _ in range(NBUF)]
        + [pltpu.VMEM_SHARED((LANES, NPLANE, 128), jnp.float32)]
        + [pltpu.SemaphoreType.DMA for _ in range(GROUPS + 2 * NBUF)]
    )
    hist = pl.kernel(
        _hist_body,
        out_type=jax.ShapeDtypeStruct((NPLANE, B, 128), jnp.float32),
        mesh=mesh,
        scratch_types=scratch,
        compiler_params=pltpu.CompilerParams(use_tc_tiling_on_sc=True,
                                             needs_layout_passes=False),
    )
    return hist(x)


def _classify_body(counts_ref, emb_ref, W_ref, b_ref, out_ref):
    pooled = jnp.dot(counts_ref[0], emb_ref[0],
                     preferred_element_type=jnp.float32)
    for j in range(1, NPLANE):
        pooled += jnp.dot(counts_ref[j], emb_ref[j],
                          preferred_element_type=jnp.float32)
    logits = lax.dot_general(pooled, W_ref[...], (((1,), (1,)), ((), ())),
                             preferred_element_type=jnp.float32)
    out_ref[...] = jax.nn.sigmoid(logits * (1.0 / L) + b_ref[...])


def _classify(counts, embp, W, b2d):
    blk = 512
    grid = (B // blk,)
    return pl.pallas_call(
        _classify_body,
        grid=grid,
        in_specs=[
            pl.BlockSpec((NPLANE, blk, 128), lambda i: (0, i, 0)),
            pl.BlockSpec((NPLANE, 128, D), lambda i: (0, 0, 0)),
            pl.BlockSpec((OUT, D), lambda i: (0, 0)),
            pl.BlockSpec((1, OUT), lambda i: (0, 0)),
        ],
        out_specs=pl.BlockSpec((blk, OUT), lambda i: (i, 0)),
        out_shape=jax.ShapeDtypeStruct((B, OUT), jnp.float32),
    )(counts, embp, W, b2d)


def kernel(x, emb, W, b):
    counts = _histogram(x)
    embp = jnp.concatenate(
        [emb, jnp.zeros((VPAD - VOCAB, D), emb.dtype)]
    ).reshape(NPLANE, 128, D)
    return _classify(counts, embp, W, b.reshape(1, OUT))


# speedup vs baseline: 1.1142x; 1.1142x over previous
import functools

import jax
import jax.numpy as jnp
import numpy as np
from jax import lax
from jax.experimental import pallas as pl
from jax.experimental.pallas import tpu as pltpu
from jax.experimental.pallas import tpu_sc as plsc

VOCAB = 1000
D = 128
L = 200
B = 4096
OUT = L + 1

VPAD = 1024
NPLANE = VPAD // 128

NC = 2
NS = 16
LANES = 16
NW = NC * NS
SEQ_PER_W = B // NW
GROUPS = SEQ_PER_W // LANES
NBUF = 4


def _hist_body(x_hbm, counts_hbm, *refs):
    x_v = refs[0:GROUPS]
    counts_v = refs[GROUPS:GROUPS + NBUF]
    zeros_sp = refs[GROUPS + NBUF]
    n0 = GROUPS + NBUF + 1
    sx = refs[n0:n0 + GROUPS]
    sz = refs[n0 + GROUPS:n0 + GROUPS + NBUF]
    so = refs[n0 + GROUPS + NBUF:n0 + GROUPS + 2 * NBUF]

    cid = lax.axis_index("c")
    sid = lax.axis_index("s")
    wid = sid * NC + cid
    row_ids = lax.iota(jnp.int32, LANES)
    ones = jnp.ones((LANES,), jnp.float32)

    zslab = counts_v[0]

    def zero_body(it, zslab=zslab):
        j = lax.shift_right_logical(it, 3)
        c = lax.shift_left(lax.bitwise_and(it, 7), 4)
        zslab[0, j, pl.ds(c, LANES)] = jnp.zeros((LANES,), jnp.float32)

    plsc.parallel_loop(0, NPLANE * 8, step=1, unroll=8)(zero_body)
    pltpu.sync_copy(zslab.at[0], zeros_sp.at[sid])
    plsc.subcore_barrier()

    def x_copy(g):
        base = wid * SEQ_PER_W + g * LANES
        return pltpu.make_async_copy(
            x_hbm.at[pl.ds(base, LANES), :], x_v[g], sx[g])

    def z_copy(b):
        return pltpu.make_async_copy(zeros_sp, counts_v[b], sz[b])

    def out_copy(g, b, j):
        base = wid * SEQ_PER_W + g * LANES
        return pltpu.make_async_copy(
            counts_v[b].at[:, j, :],
            counts_hbm.at[j, pl.ds(base, LANES), :], so[b])

    for g in range(GROUPS):
        x_copy(g).start()
    z_copy(0).start()

    for g in range(GROUPS):
        b = g % NBUF
        if g + 1 < GROUPS:
            b2 = (g + 1) % NBUF
            if g + 1 >= NBUF:
                for j in range(NPLANE):
                    out_copy(g + 1 - NBUF, b2, j).wait()
            z_copy(b2).start()
        x_copy(g).wait()
        z_copy(b).wait()

        x_b = x_v[g]
        c_b = counts_v[b]

        def tok_body(l, x_b=x_b, c_b=c_b):
            col = jnp.full((LANES,), l, dtype=jnp.int32)
            toks = plsc.load_gather(x_b, [row_ids, col])
            plsc.addupdate_scatter(
                c_b,
                [row_ids, lax.shift_right_logical(toks, 7),
                 lax.bitwise_and(toks, 127)],
                ones)

        plsc.parallel_loop(0, L, step=1, unroll=8)(tok_body)

        for j in range(NPLANE):
            out_copy(g, b, j).start()
    for g in range(GROUPS - NBUF, GROUPS):
        for j in range(NPLANE):
            out_copy(g, g % NBUF, j).wait()


def _histogram(x):
    mesh = plsc.VectorSubcoreMesh(core_axis_name="c", subcore_axis_name="s")
    scratch = (
        [pltpu.VMEM((LANES, L), jnp.int32) for _ in range(GROUPS)]
        + [pltpu.VMEM((LANES, NPLANE, 128), jnp.float32) for _ in range(NBUF)]
        + [pltpu.VMEM_SHARED((LANES, NPLANE, 128), jnp.float32)]
        + [pltpu.SemaphoreType.DMA for _ in range(GROUPS + 2 * NBUF)]
    )
    hist = pl.kernel(
        _hist_body,
        out_type=jax.ShapeDtypeStruct((NPLANE, B, 128), jnp.float32),
        mesh=mesh,
        scratch_types=scratch,
        compiler_params=pltpu.CompilerParams(use_tc_tiling_on_sc=False,
                                             needs_layout_passes=False),
    )
    return hist(x)


def _classify_body(counts_ref, emb_ref, W_ref, b_ref, out_ref):
    pooled = jnp.dot(counts_ref[0], emb_ref[0],
                     preferred_element_type=jnp.float32)
    for j in range(1, NPLANE):
        pooled += jnp.dot(counts_ref[j], emb_ref[j],
                          preferred_element_type=jnp.float32)
    logits = lax.dot_general(pooled, W_ref[...], (((1,), (1,)), ((), ())),
                             preferred_element_type=jnp.float32)
    out_ref[...] = jax.nn.sigmoid(logits * (1.0 / L) + b_ref[...])


def _classify(counts, embp, W, b2d):
    blk = 1024
    grid = (B // blk,)
    return pl.pallas_call(
        _classify_body,
        grid=grid,
        in_specs=[
            pl.BlockSpec((NPLANE, blk, 128), lambda i: (0, i, 0)),
            pl.BlockSpec((NPLANE, 128, D), lambda i: (0, 0, 0)),
            pl.BlockSpec((OUT, D), lambda i: (0, 0)),
            pl.BlockSpec((1, OUT), lambda i: (0, 0)),
        ],
        out_specs=pl.BlockSpec((blk, OUT), lambda i: (i, 0)),
        out_shape=jax.ShapeDtypeStruct((B, OUT), jnp.float32),
    )(counts, embp, W, b2d)


def kernel(x, emb, W, b):
    counts = _histogram(x)
    embp = jnp.concatenate(
        [emb, jnp.zeros((VPAD - VOCAB, D), emb.dtype)]
    ).reshape(NPLANE, 128, D)
    return _classify(counts, embp, W, b.reshape(1, OUT))


# speedup vs baseline: 1.1170x; 1.0026x over previous
import functools

import jax
import jax.numpy as jnp
import numpy as np
from jax import lax
from jax.experimental import pallas as pl
from jax.experimental.pallas import tpu as pltpu
from jax.experimental.pallas import tpu_sc as plsc

VOCAB = 1000
D = 128
L = 200
B = 4096
OUT = L + 1

VPAD = 1024
NPLANE = VPAD // 128

NC = 2
NS = 16
LANES = 16
NW = NC * NS
SEQ_PER_W = B // NW
GROUPS = SEQ_PER_W // LANES
NBUF = 4


def _hist_body(x_hbm, counts_hbm, *refs):
    x_v = refs[0:GROUPS]
    counts_v = refs[GROUPS:GROUPS + NBUF]
    zeros_sp = refs[GROUPS + NBUF]
    n0 = GROUPS + NBUF + 1
    sx = refs[n0:n0 + GROUPS]
    sz = refs[n0 + GROUPS:n0 + GROUPS + NBUF]
    so = refs[n0 + GROUPS + NBUF:n0 + GROUPS + 2 * NBUF]

    cid = lax.axis_index("c")
    sid = lax.axis_index("s")
    wid = sid * NC + cid
    row_ids = lax.iota(jnp.int32, LANES)
    row_off = row_ids * L
    ones = jnp.ones((LANES,), jnp.float32)

    zslab = counts_v[0]

    def zero_body(it, zslab=zslab):
        j = lax.shift_right_logical(it, 3)
        c = lax.shift_left(lax.bitwise_and(it, 7), 4)
        zslab[0, j, pl.ds(c, LANES)] = jnp.zeros((LANES,), jnp.float32)

    plsc.parallel_loop(0, NPLANE * 8, step=1, unroll=8)(zero_body)
    pltpu.sync_copy(zslab.at[0], zeros_sp.at[sid])
    plsc.subcore_barrier()

    def x_copy(g):
        base = (wid * SEQ_PER_W + g * LANES) * L
        return pltpu.make_async_copy(
            x_hbm.at[pl.ds(base, LANES * L)], x_v[g], sx[g])

    def z_copy(b):
        return pltpu.make_async_copy(zeros_sp, counts_v[b], sz[b])

    def out_copy(g, b, j):
        base = wid * SEQ_PER_W + g * LANES
        return pltpu.make_async_copy(
            counts_v[b].at[:, j, :],
            counts_hbm.at[j, pl.ds(base, LANES), :], so[b])

    for g in range(GROUPS):
        x_copy(g).start()
    z_copy(0).start()

    for g in range(GROUPS):
        b = g % NBUF
        if g + 1 < GROUPS:
            b2 = (g + 1) % NBUF
            if g + 1 >= NBUF:
                for j in range(NPLANE):
                    out_copy(g + 1 - NBUF, b2, j).wait()
            z_copy(b2).start()
        x_copy(g).wait()
        z_copy(b).wait()

        x_b = x_v[g]
        c_b = counts_v[b]

        def tok_body(l, x_b=x_b, c_b=c_b):
            idx = row_off + l
            toks = plsc.load_gather(x_b, [idx])
            plsc.addupdate_scatter(
                c_b,
                [row_ids, lax.shift_right_logical(toks, 7),
                 lax.bitwise_and(toks, 127)],
                ones)

        plsc.parallel_loop(0, L, step=1, unroll=8)(tok_body)

        for j in range(NPLANE):
            out_copy(g, b, j).start()
    for g in range(GROUPS - NBUF, GROUPS):
        for j in range(NPLANE):
            out_copy(g, g % NBUF, j).wait()


def _histogram(x):
    mesh = plsc.VectorSubcoreMesh(core_axis_name="c", subcore_axis_name="s")
    scratch = (
        [pltpu.VMEM((LANES * L,), jnp.int32) for _ in range(GROUPS)]
        + [pltpu.VMEM((LANES, NPLANE, 128), jnp.float32) for _ in range(NBUF)]
        + [pltpu.VMEM_SHARED((LANES, NPLANE, 128), jnp.float32)]
        + [pltpu.SemaphoreType.DMA for _ in range(GROUPS + 2 * NBUF)]
    )
    hist = pl.kernel(
        _hist_body,
        out_type=jax.ShapeDtypeStruct((NPLANE, B, 128), jnp.float32),
        mesh=mesh,
        scratch_types=scratch,
        compiler_params=pltpu.CompilerParams(use_tc_tiling_on_sc=False,
                                             needs_layout_passes=False),
    )
    return hist(x.reshape(B * L))


def _classify_body(counts_ref, emb_ref, W_ref, b_ref, out_ref):
    pooled = jnp.dot(counts_ref[0], emb_ref[0],
                     preferred_element_type=jnp.float32)
    for j in range(1, NPLANE):
        pooled += jnp.dot(counts_ref[j], emb_ref[j],
                          preferred_element_type=jnp.float32)
    logits = lax.dot_general(pooled, W_ref[...], (((1,), (1,)), ((), ())),
                             preferred_element_type=jnp.float32)
    out_ref[...] = jax.nn.sigmoid(logits * (1.0 / L) + b_ref[...])


def _classify(counts, embp, W, b2d):
    blk = 1024
    grid = (B // blk,)
    return pl.pallas_call(
        _classify_body,
        grid=grid,
        in_specs=[
            pl.BlockSpec((NPLANE, blk, 128), lambda i: (0, i, 0)),
            pl.BlockSpec((NPLANE, 128, D), lambda i: (0, 0, 0)),
            pl.BlockSpec((OUT, D), lambda i: (0, 0)),
            pl.BlockSpec((1, OUT), lambda i: (0, 0)),
        ],
        out_specs=pl.BlockSpec((blk, OUT), lambda i: (i, 0)),
        out_shape=jax.ShapeDtypeStruct((B, OUT), jnp.float32),
    )(counts, embp, W, b2d)


def kernel(x, emb, W, b):
    counts = _histogram(x)
    embp = jnp.concatenate(
        [emb, jnp.zeros((VPAD - VOCAB, D), emb.dtype)]
    ).reshape(NPLANE, 128, D)
    return _classify(counts, embp, W, b.reshape(1, OUT))
